# baseline (speedup 1.0000x reference)
import jax
import jax.numpy as jnp
import numpy as np
from jax.experimental import pallas as pl
from jax.experimental.pallas import tpu as pltpu

_N = 16384
_D = 256
_K = 256
_B = 1024
_CSCALE = 2.0 / (_N * _D)
_SSCALE = 1.0 / (_N * 4 * _D)


def _make_adj():
    idx = np.arange(_K)
    i1, i2 = idx // 16, idx % 16
    adj = np.zeros((_K, _K), np.float32)
    for nbi in (np.clip(i1 - 1, 0, 15) * 16 + i2, np.clip(i1 + 1, 0, 15) * 16 + i2,
                i1 * 16 + np.clip(i2 - 1, 0, 15), i1 * 16 + np.clip(i2 + 1, 0, 15)):
        np.add.at(adj, (idx, nbi), 1.0)
    return adj


_ADJ = _make_adj()


def _lrelu(v):
    return jnp.maximum(v, 0.01 * v)


def _body(x_ref, eps_ref, vecs_ref, mats_ref, big_ref,
          ze_ref, zq_ref, de_ref, dq_ref, cs_ref, ss_ref, tbl_ref):
    w0 = vecs_ref[:, 0:128]
    b0 = vecs_ref[:, 128:256]
    b1 = vecs_ref[:, 256:384]
    bd = vecs_ref[:, 384:512]
    bd0 = vecs_ref[:, 512:640]
    bd1 = vecs_ref[:, 640:768]
    bd2 = vecs_ref[:, 768:896]
    bml = vecs_ref[:, 896:1408]
    e2 = vecs_ref[:, 1408:1664]
    W1 = mats_ref[:, 0:128]
    Wml = mats_ref[:, 128:640]
    Wd0 = mats_ref[:, 640:768]
    Wd1 = mats_ref[:, 768:896]
    Wd2 = mats_ref[:, 896:1024]
    embT = big_ref[:, 0:256]
    embA = big_ref[:, 256:768]
    Wd = big_ref[:, 768:896]

    def dec(z):
        y = _lrelu(jnp.dot(z, Wd) + bd)
        y = _lrelu(jnp.dot(y, Wd0) + bd0)
        y = _lrelu(jnp.dot(y, Wd1) + bd1)
        y = _lrelu(jnp.dot(y, Wd2) + bd2)
        return y

    @pl.when(pl.program_id(0) == 0)
    def _mk_table():
        tbl_ref[...] = dec(big_ref[:, 256:512])

    xb = x_ref[...]
    h = _lrelu(xb * w0 + b0)
    h = _lrelu(jnp.dot(h, W1) + b1)
    ml = jnp.dot(h, Wml) + bml
    mu, lv = ml[:, :_D], ml[:, _D:]
    ze = mu + eps_ref[...] * jnp.exp(0.5 * lv)
    ze_ref[...] = ze

    dots = jnp.dot(ze, embT)
    z2 = jnp.sum(ze * ze, axis=1, keepdims=True)
    d = (z2 - 2.0 * dots) + e2
    dmin = jnp.min(d, axis=1, keepdims=True)
    j = jax.lax.broadcasted_iota(jnp.int32, d.shape, 1)
    k = jnp.min(jnp.where(d == dmin, j, _K), axis=1, keepdims=True)

    cs_part = jnp.sum(dmin) * _CSCALE

    oh = (j == k).astype(jnp.float32)
    ga = jnp.dot(oh, embA)
    zq = ga[:, :_D]
    m = ga[:, _D:]
    ss_part = jnp.sum(m * d) * _SSCALE
    zq_ref[...] = zq

    de_ref[...] = dec(ze)[:, 0:1]
    dq_ref[...] = jnp.dot(oh, tbl_ref[...])[:, 0:1]

    @pl.when(pl.program_id(0) == 0)
    def _init():
        cs_ref[...] = jnp.zeros_like(cs_ref)
        ss_ref[...] = jnp.zeros_like(ss_ref)

    cs_ref[...] += cs_part
    ss_ref[...] += ss_part


def kernel(x, W_e0, b_e0, W_e1, b_e1, W_mu, b_mu, W_lv, b_lv,
           W_d, b_d, W_d0, b_d0, W_d1, b_d1, W_d2, b_d2, emb, eps):
    f32 = jnp.float32
    vecs = (jnp.zeros((1, 1664), f32)
            .at[0, 0:10].set(W_e0[:, 0])
            .at[0, 128:138].set(b_e0)
            .at[0, 256:306].set(b_e1)
            .at[0, 384:484].set(b_d)
            .at[0, 512:572].set(b_d0)
            .at[0, 640:670].set(b_d1)
            .at[0, 768:769].set(b_d2)
            .at[0, 896:1152].set(b_mu)
            .at[0, 1152:1408].set(b_lv)
            .at[0, 1408:1664].set(jnp.sum(emb * emb, axis=1)))
    mats = (jnp.zeros((128, 1024), f32)
            .at[:10, 0:50].set(W_e1.T)
            .at[:50, 128:384].set(W_mu.T)
            .at[:50, 384:640].set(W_lv.T)
            .at[:100, 640:700].set(W_d0.T)
            .at[:60, 768:798].set(W_d1.T)
            .at[:30, 896:897].set(W_d2.T))
    big = (jnp.zeros((256, 896), f32)
           .at[:, 0:256].set(emb.T)
           .at[:, 256:512].set(emb)
           .at[:, 512:768].set(jnp.asarray(_ADJ))
           .at[:, 768:868].set(W_d.T))

    full = lambda shape: pl.BlockSpec(shape, lambda i: (0, 0))
    rows = lambda cols: pl.BlockSpec((_B, cols), lambda i: (i, 0))

    ze, zq, de, dq, cs, ss = pl.pallas_call(
        _body,
        grid=(_N // _B,),
        in_specs=[
            rows(1), rows(_D),
            full((1, 1664)), full((128, 1024)), full((256, 896)),
        ],
        out_specs=[
            rows(_D), rows(_D), rows(1), rows(1),
            pl.BlockSpec((1, 1), lambda i: (0, 0)),
            pl.BlockSpec((1, 1), lambda i: (0, 0)),
        ],
        out_shape=[
            jax.ShapeDtypeStruct((_N, _D), f32),
            jax.ShapeDtypeStruct((_N, _D), f32),
            jax.ShapeDtypeStruct((_N, 1), f32),
            jax.ShapeDtypeStruct((_N, 1), f32),
            jax.ShapeDtypeStruct((1, 1), f32),
            jax.ShapeDtypeStruct((1, 1), f32),
        ],
        scratch_shapes=[pltpu.VMEM((_K, 128), f32)],
    )(x, eps, vecs, mats, big)

    return ze, zq, de, dq, cs[0, 0], ss[0, 0]


# speedup vs baseline: 1.1791x; 1.1791x over previous
import jax
import jax.numpy as jnp
import numpy as np
from jax.experimental import pallas as pl
from jax.experimental.pallas import tpu as pltpu

_N = 16384
_D = 256
_K = 256
_B = 512
_CSCALE = 2.0 / (_N * _D)
_SSCALE = 1.0 / (_N * 4 * _D)


def _make_adj():
    idx = np.arange(_K)
    i1, i2 = idx // 16, idx % 16
    adj = np.zeros((_K, _K), np.float32)
    for nbi in (np.clip(i1 - 1, 0, 15) * 16 + i2, np.clip(i1 + 1, 0, 15) * 16 + i2,
                i1 * 16 + np.clip(i2 - 1, 0, 15), i1 * 16 + np.clip(i2 + 1, 0, 15)):
        np.add.at(adj, (idx, nbi), 1.0)
    return adj


_ADJ = _make_adj()


def _lrelu(v):
    return jnp.maximum(v, 0.01 * v)


def _body(x_ref, eps_ref, w0_ref, b0_ref, W1_ref, b1_ref, Wml_ref, bml_ref,
          embT_ref, embA_ref, e2_ref,
          Wd_ref, bd_ref, Wd0_ref, bd0_ref, Wd1_ref, bd1_ref, Wd2_ref, bd2_ref,
          ze_ref, zq_ref, de_ref, dq_ref, cs_ref, ss_ref, tbl_ref):
    def dec(z):
        y = _lrelu(jnp.dot(z, Wd_ref[...]) + bd_ref[...])
        y = _lrelu(jnp.dot(y, Wd0_ref[...]) + bd0_ref[...])
        y = _lrelu(jnp.dot(y, Wd1_ref[...]) + bd1_ref[...])
        y = _lrelu(jnp.dot(y, Wd2_ref[...]) + bd2_ref[...])
        return y

    @pl.when(pl.program_id(0) == 0)
    def _mk_table():
        tbl_ref[...] = dec(embA_ref[:, :_D])

    xb = x_ref[...]
    h = _lrelu(xb * w0_ref[...] + b0_ref[...])
    h = _lrelu(jnp.dot(h, W1_ref[...]) + b1_ref[...])
    ml = jnp.dot(h, Wml_ref[...]) + bml_ref[...]
    mu, lv = ml[:, :_D], ml[:, _D:]
    ze = mu + eps_ref[...] * jnp.exp(0.5 * lv)
    ze_ref[...] = ze

    dots = jnp.dot(ze, embT_ref[...])
    z2 = jnp.sum(ze * ze, axis=1, keepdims=True)
    d = (z2 - 2.0 * dots) + e2_ref[...]
    dmin = jnp.min(d, axis=1, keepdims=True)
    j = jax.lax.broadcasted_iota(jnp.int32, d.shape, 1)
    k = jnp.min(jnp.where(d == dmin, j, _K), axis=1, keepdims=True)

    cs_part = jnp.sum(dmin) * _CSCALE

    oh = (j == k).astype(jnp.float32)
    ga = jnp.dot(oh, embA_ref[...])
    zq = ga[:, :_D]
    m = ga[:, _D:]
    ss_part = jnp.sum(m * d) * _SSCALE
    zq_ref[...] = zq

    de_ref[...] = dec(ze)[:, 0:1]
    dq_ref[...] = jnp.dot(oh, tbl_ref[...])[:, 0:1]

    @pl.when(pl.program_id(0) == 0)
    def _init():
        cs_ref[...] = jnp.zeros_like(cs_ref)
        ss_ref[...] = jnp.zeros_like(ss_ref)

    cs_ref[...] += cs_part
    ss_ref[...] += ss_part


def kernel(x, W_e0, b_e0, W_e1, b_e1, W_mu, b_mu, W_lv, b_lv,
           W_d, b_d, W_d0, b_d0, W_d1, b_d1, W_d2, b_d2, emb, eps):
    f32 = jnp.float32
    w0p = jnp.zeros((1, 128), f32).at[0, :10].set(W_e0[:, 0])
    b0p = jnp.zeros((1, 128), f32).at[0, :10].set(b_e0)
    W1p = jnp.zeros((128, 128), f32).at[:10, :50].set(W_e1.T)
    b1p = jnp.zeros((1, 128), f32).at[0, :50].set(b_e1)
    Wmlp = (jnp.zeros((128, 2 * _D), f32)
            .at[:50, :_D].set(W_mu.T).at[:50, _D:].set(W_lv.T))
    bmlp = jnp.concatenate([b_mu, b_lv]).reshape(1, 2 * _D)
    embT = emb.T
    e2 = jnp.sum(emb * emb, axis=1).reshape(1, _K)
    embA = jnp.concatenate([emb, jnp.asarray(_ADJ)], axis=1)
    Wdp = jnp.zeros((_D, 128), f32).at[:, :100].set(W_d.T)
    bdp = jnp.zeros((1, 128), f32).at[0, :100].set(b_d)
    Wd0p = jnp.zeros((128, 128), f32).at[:100, :60].set(W_d0.T)
    bd0p = jnp.zeros((1, 128), f32).at[0, :60].set(b_d0)
    Wd1p = jnp.zeros((128, 128), f32).at[:60, :30].set(W_d1.T)
    bd1p = jnp.zeros((1, 128), f32).at[0, :30].set(b_d1)
    Wd2p = jnp.zeros((128, 128), f32).at[:30, :1].set(W_d2.T)
    bd2p = jnp.zeros((1, 128), f32).at[0, 0].set(b_d2[0])

    full = lambda shape: pl.BlockSpec(shape, lambda i: (0, 0))
    rows = lambda cols: pl.BlockSpec((_B, cols), lambda i: (i, 0))

    ze, zq, de, dq, cs, ss = pl.pallas_call(
        _body,
        grid=(_N // _B,),
        in_specs=[
            rows(1), rows(_D),
            full((1, 128)), full((1, 128)), full((128, 128)), full((1, 128)),
            full((128, 2 * _D)), full((1, 2 * _D)),
            full((_D, _K)), full((_K, 2 * _K)), full((1, _K)),
            full((_D, 128)), full((1, 128)), full((128, 128)), full((1, 128)),
            full((128, 128)), full((1, 128)), full((128, 128)), full((1, 128)),
        ],
        out_specs=[
            rows(_D), rows(_D), rows(1), rows(1),
            pl.BlockSpec((1, 1), lambda i: (0, 0)),
            pl.BlockSpec((1, 1), lambda i: (0, 0)),
        ],
        out_shape=[
            jax.ShapeDtypeStruct((_N, _D), f32),
            jax.ShapeDtypeStruct((_N, _D), f32),
            jax.ShapeDtypeStruct((_N, 1), f32),
            jax.ShapeDtypeStruct((_N, 1), f32),
            jax.ShapeDtypeStruct((1, 1), f32),
            jax.ShapeDtypeStruct((1, 1), f32),
        ],
        scratch_shapes=[pltpu.VMEM((_K, 128), f32)],
    )(x, eps, w0p, b0p, W1p, b1p, Wmlp, bmlp, embT, embA, e2,
      Wdp, bdp, Wd0p, bd0p, Wd1p, bd1p, Wd2p, bd2p)

    return ze, zq, de, dq, cs[0, 0], ss[0, 0]


# speedup vs baseline: 2.0278x; 1.7197x over previous
import jax
import jax.numpy as jnp
import numpy as np
from jax.experimental import pallas as pl
from jax.experimental.pallas import tpu as pltpu

_N = 16384
_D = 256
_K = 256
_B = 1024
_CSCALE = 2.0 / (_N * _D)
_SSCALE = 1.0 / (_N * 4 * _D)


def _make_adj():
    idx = np.arange(_K)
    i1, i2 = idx // 16, idx % 16
    adj = np.zeros((_K, _K), np.float32)
    for nbi in (np.clip(i1 - 1, 0, 15) * 16 + i2, np.clip(i1 + 1, 0, 15) * 16 + i2,
                i1 * 16 + np.clip(i2 - 1, 0, 15), i1 * 16 + np.clip(i2 + 1, 0, 15)):
        np.add.at(adj, (idx, nbi), 1.0)
    return adj


_ADJ = _make_adj()


def _lrelu(v):
    return jnp.maximum(v, 0.01 * v)


def _dott(a, w):
    return jax.lax.dot_general(a, w, (((1,), (1,)), ((), ())))


def _body(x_ref, eps_ref, w0_ref, b0_ref, W1_ref, b1_ref, Wmu_ref, bmu_ref,
          Wlv_ref, blv_ref, emb_ref, e2_ref, adj_ref,
          Wd_ref, bd_ref, Wd0_ref, bd0_ref, Wd1_ref, bd1_ref, Wd2_ref, bd2_ref,
          ze_ref, zq_ref, de_ref, dq_ref, cs_ref, ss_ref,
          w0s, b0s, W1s, b1s, Wmls, bmls,
          Wds, bds, Wd0s, bd0s, Wd1s, bd1s, Wd2s, bd2s, tbl_ref):

    def dec(z):
        y = _lrelu(_dott(z, Wds[...]) + bds[...])
        y = _lrelu(_dott(y, Wd0s[...]) + bd0s[...])
        y = _lrelu(_dott(y, Wd1s[...]) + bd1s[...])
        y = _lrelu(_dott(y, Wd2s[...]) + bd2s[...])
        return y

    @pl.when(pl.program_id(0) == 0)
    def _prep():
        w0s[...] = jnp.zeros_like(w0s)
        w0s[0:1, 0:10] = w0_ref[...]
        b0s[...] = jnp.zeros_like(b0s)
        b0s[0:1, 0:10] = b0_ref[...]
        W1s[...] = jnp.zeros_like(W1s)
        W1s[0:50, 0:10] = W1_ref[...]
        b1s[...] = jnp.zeros_like(b1s)
        b1s[0:1, 0:50] = b1_ref[...]
        Wmls[...] = jnp.zeros_like(Wmls)
        Wmls[0:256, 0:50] = Wmu_ref[...]
        Wmls[256:512, 0:50] = Wlv_ref[...]
        bmls[0:1, 0:256] = bmu_ref[...]
        bmls[0:1, 256:512] = blv_ref[...]
        Wds[...] = jnp.zeros_like(Wds)
        Wds[0:100, :] = Wd_ref[...]
        bds[...] = jnp.zeros_like(bds)
        bds[0:1, 0:100] = bd_ref[...]
        Wd0s[...] = jnp.zeros_like(Wd0s)
        Wd0s[0:60, 0:100] = Wd0_ref[...]
        bd0s[...] = jnp.zeros_like(bd0s)
        bd0s[0:1, 0:60] = bd0_ref[...]
        Wd1s[...] = jnp.zeros_like(Wd1s)
        Wd1s[0:30, 0:60] = Wd1_ref[...]
        bd1s[...] = jnp.zeros_like(bd1s)
        bd1s[0:1, 0:30] = bd1_ref[...]
        Wd2s[...] = jnp.zeros_like(Wd2s)
        Wd2s[0:1, 0:30] = Wd2_ref[...]
        bd2s[...] = jnp.zeros_like(bd2s)
        bd2s[0:1, 0:1] = bd2_ref[...]
        tbl_ref[...] = dec(emb_ref[...])

    xb = x_ref[...]
    h = _lrelu(xb * w0s[...] + b0s[...])
    h = _lrelu(_dott(h, W1s[...]) + b1s[...])
    ml = _dott(h, Wmls[...]) + bmls[...]
    mu, lv = ml[:, :_D], ml[:, _D:]
    ze = mu + eps_ref[...] * jnp.exp(0.5 * lv)
    ze_ref[...] = ze

    dots = _dott(ze, emb_ref[...])
    z2 = jnp.sum(ze * ze, axis=1, keepdims=True)
    d = (z2 - 2.0 * dots) + e2_ref[...]
    dmin = jnp.min(d, axis=1, keepdims=True)
    j = jax.lax.broadcasted_iota(jnp.int32, d.shape, 1)
    k = jnp.min(jnp.where(d == dmin, j, _K), axis=1, keepdims=True)

    cs_part = jnp.sum(dmin) * _CSCALE

    oh = (j == k).astype(jnp.float32)
    zq = jnp.dot(oh, emb_ref[...])
    m = jnp.dot(oh, adj_ref[...])
    ss_part = jnp.sum(m * d) * _SSCALE
    zq_ref[...] = zq

    de_ref[...] = dec(ze)[:, 0:1]
    dq_ref[...] = jnp.dot(oh, tbl_ref[...])[:, 0:1]

    @pl.when(pl.program_id(0) == 0)
    def _init():
        cs_ref[...] = jnp.zeros_like(cs_ref)
        ss_ref[...] = jnp.zeros_like(ss_ref)

    cs_ref[...] += cs_part
    ss_ref[...] += ss_part


def kernel(x, W_e0, b_e0, W_e1, b_e1, W_mu, b_mu, W_lv, b_lv,
           W_d, b_d, W_d0, b_d0, W_d1, b_d1, W_d2, b_d2, emb, eps):
    e2 = jnp.sum(emb * emb, axis=1).reshape(1, _K)

    full = lambda shape: pl.BlockSpec(shape, lambda i: (0, 0))
    rows = lambda cols: pl.BlockSpec((_B, cols), lambda i: (i, 0))
    f32 = jnp.float32
    vmem = lambda shape: pltpu.VMEM(shape, f32)

    ze, zq, de, dq, cs, ss = pl.pallas_call(
        _body,
        grid=(_N // _B,),
        in_specs=[
            rows(1), rows(_D),
            full((1, 10)), full((1, 10)), full((50, 10)), full((1, 50)),
            full((_D, 50)), full((1, _D)), full((_D, 50)), full((1, _D)),
            full((_K, _D)), full((1, _K)), full((_K, _K)),
            full((100, _D)), full((1, 100)), full((60, 100)), full((1, 60)),
            full((30, 60)), full((1, 30)), full((1, 30)), full((1, 1)),
        ],
        out_specs=[
            rows(_D), rows(_D), rows(1), rows(1),
            pl.BlockSpec((1, 1), lambda i: (0, 0)),
            pl.BlockSpec((1, 1), lambda i: (0, 0)),
        ],
        out_shape=[
            jax.ShapeDtypeStruct((_N, _D), f32),
            jax.ShapeDtypeStruct((_N, _D), f32),
            jax.ShapeDtypeStruct((_N, 1), f32),
            jax.ShapeDtypeStruct((_N, 1), f32),
            jax.ShapeDtypeStruct((1, 1), f32),
            jax.ShapeDtypeStruct((1, 1), f32),
        ],
        scratch_shapes=[
            vmem((1, 128)), vmem((1, 128)), vmem((128, 128)), vmem((1, 128)),
            vmem((512, 128)), vmem((1, 512)),
            vmem((128, _D)), vmem((1, 128)), vmem((128, 128)), vmem((1, 128)),
            vmem((128, 128)), vmem((1, 128)), vmem((128, 128)), vmem((1, 128)),
            vmem((_K, 128)),
        ],
    )(x, eps,
      W_e0.reshape(1, 10), b_e0.reshape(1, 10), W_e1, b_e1.reshape(1, 50),
      W_mu, b_mu.reshape(1, _D), W_lv, b_lv.reshape(1, _D),
      emb, e2, jnp.asarray(_ADJ),
      W_d, b_d.reshape(1, 100), W_d0, b_d0.reshape(1, 60),
      W_d1, b_d1.reshape(1, 30), W_d2, b_d2.reshape(1, 1))

    return ze, zq, de, dq, cs[0, 0], ss[0, 0]


# speedup vs baseline: 2.1695x; 1.0699x over previous
import jax
import jax.numpy as jnp
import numpy as np
from jax.experimental import pallas as pl
from jax.experimental.pallas import tpu as pltpu

_N = 16384
_D = 256
_K = 256
_B = 2048
_CSCALE = 2.0 / (_N * _D)
_SSCALE = 1.0 / (_N * 4 * _D)


def _make_adj():
    idx = np.arange(_K)
    i1, i2 = idx // 16, idx % 16
    adj = np.zeros((_K, _K), np.float32)
    for nbi in (np.clip(i1 - 1, 0, 15) * 16 + i2, np.clip(i1 + 1, 0, 15) * 16 + i2,
                i1 * 16 + np.clip(i2 - 1, 0, 15), i1 * 16 + np.clip(i2 + 1, 0, 15)):
        np.add.at(adj, (idx, nbi), 1.0)
    return adj


_ADJ = _make_adj()


def _lrelu(v):
    return jnp.maximum(v, 0.01 * v)


def _dott(a, w):
    return jax.lax.dot_general(a, w, (((1,), (1,)), ((), ())))


def _body(x_ref, eps_ref, w0_ref, b0_ref, W1_ref, b1_ref, Wmu_ref, bmu_ref,
          Wlv_ref, blv_ref, emb_ref, e2_ref, adj_ref,
          Wd_ref, bd_ref, Wd0_ref, bd0_ref, Wd1_ref, bd1_ref, Wd2_ref, bd2_ref,
          ze_ref, zq_ref, de_ref, dq_ref, cs_ref, ss_ref,
          w0s, b0s, W1s, b1s, Wmls, bmls,
          Wds, bds, Wd0s, bd0s, Wd1s, bd1s, Wd2s, bd2s, tbl_ref):

    def dec(z):
        y = _lrelu(_dott(z, Wds[...]) + bds[...])
        y = _lrelu(_dott(y, Wd0s[...]) + bd0s[...])
        y = _lrelu(_dott(y, Wd1s[...]) + bd1s[...])
        y = _lrelu(_dott(y, Wd2s[...]) + bd2s[...])
        return y

    @pl.when(pl.program_id(0) == 0)
    def _prep():
        w0s[...] = jnp.zeros_like(w0s)
        w0s[0:1, 0:10] = w0_ref[...]
        b0s[...] = jnp.zeros_like(b0s)
        b0s[0:1, 0:10] = b0_ref[...]
        W1s[...] = jnp.zeros_like(W1s)
        W1s[0:50, 0:10] = W1_ref[...]
        b1s[...] = jnp.zeros_like(b1s)
        b1s[0:1, 0:50] = b1_ref[...]
        Wmls[...] = jnp.zeros_like(Wmls)
        Wmls[0:256, 0:50] = Wmu_ref[...]
        Wmls[256:512, 0:50] = Wlv_ref[...]
        bmls[0:1, 0:256] = bmu_ref[...]
        bmls[0:1, 256:512] = blv_ref[...]
        Wds[...] = jnp.zeros_like(Wds)
        Wds[0:100, :] = Wd_ref[...]
        bds[...] = jnp.zeros_like(bds)
        bds[0:1, 0:100] = bd_ref[...]
        Wd0s[...] = jnp.zeros_like(Wd0s)
        Wd0s[0:60, 0:100] = Wd0_ref[...]
        bd0s[...] = jnp.zeros_like(bd0s)
        bd0s[0:1, 0:60] = bd0_ref[...]
        Wd1s[...] = jnp.zeros_like(Wd1s)
        Wd1s[0:30, 0:60] = Wd1_ref[...]
        bd1s[...] = jnp.zeros_like(bd1s)
        bd1s[0:1, 0:30] = bd1_ref[...]
        Wd2s[...] = jnp.zeros_like(Wd2s)
        Wd2s[0:1, 0:30] = Wd2_ref[...]
        bd2s[...] = jnp.zeros_like(bd2s)
        bd2s[0:1, 0:1] = bd2_ref[...]
        tbl_ref[...] = dec(emb_ref[...])

    xb = x_ref[...]
    h = _lrelu(xb * w0s[...] + b0s[...])
    h = _lrelu(_dott(h, W1s[...]) + b1s[...])
    ml = _dott(h, Wmls[...]) + bmls[...]
    mu, lv = ml[:, :_D], ml[:, _D:]
    ze = mu + eps_ref[...] * jnp.exp(0.5 * lv)
    ze_ref[...] = ze

    dots = _dott(ze, emb_ref[...])
    z2 = jnp.sum(ze * ze, axis=1, keepdims=True)
    d = (z2 - 2.0 * dots) + e2_ref[...]
    dmin = jnp.min(d, axis=1, keepdims=True)
    j = jax.lax.broadcasted_iota(jnp.int32, d.shape, 1)
    k = jnp.min(jnp.where(d == dmin, j, _K), axis=1, keepdims=True)

    cs_part = jnp.sum(dmin) * _CSCALE

    oh = (j == k).astype(jnp.float32)
    zq = jnp.dot(oh, emb_ref[...])
    m = jnp.dot(oh, adj_ref[...])
    ss_part = jnp.sum(m * d) * _SSCALE
    zq_ref[...] = zq

    de_ref[...] = dec(ze)[:, 0:1]
    dq_ref[...] = jnp.dot(oh, tbl_ref[...])[:, 0:1]

    @pl.when(pl.program_id(0) == 0)
    def _init():
        cs_ref[...] = jnp.zeros_like(cs_ref)
        ss_ref[...] = jnp.zeros_like(ss_ref)

    cs_ref[...] += cs_part
    ss_ref[...] += ss_part


def kernel(x, W_e0, b_e0, W_e1, b_e1, W_mu, b_mu, W_lv, b_lv,
           W_d, b_d, W_d0, b_d0, W_d1, b_d1, W_d2, b_d2, emb, eps):
    e2 = jnp.sum(emb * emb, axis=1).reshape(1, _K)

    full = lambda shape: pl.BlockSpec(shape, lambda i: (0, 0))
    rows = lambda cols: pl.BlockSpec((_B, cols), lambda i: (i, 0))
    f32 = jnp.float32
    vmem = lambda shape: pltpu.VMEM(shape, f32)

    ze, zq, de, dq, cs, ss = pl.pallas_call(
        _body,
        grid=(_N // _B,),
        in_specs=[
            rows(1), rows(_D),
            full((1, 10)), full((1, 10)), full((50, 10)), full((1, 50)),
            full((_D, 50)), full((1, _D)), full((_D, 50)), full((1, _D)),
            full((_K, _D)), full((1, _K)), full((_K, _K)),
            full((100, _D)), full((1, 100)), full((60, 100)), full((1, 60)),
            full((30, 60)), full((1, 30)), full((1, 30)), full((1, 1)),
        ],
        out_specs=[
            rows(_D), rows(_D), rows(1), rows(1),
            pl.BlockSpec((1, 1), lambda i: (0, 0)),
            pl.BlockSpec((1, 1), lambda i: (0, 0)),
        ],
        out_shape=[
            jax.ShapeDtypeStruct((_N, _D), f32),
            jax.ShapeDtypeStruct((_N, _D), f32),
            jax.ShapeDtypeStruct((_N, 1), f32),
            jax.ShapeDtypeStruct((_N, 1), f32),
            jax.ShapeDtypeStruct((1, 1), f32),
            jax.ShapeDtypeStruct((1, 1), f32),
        ],
        scratch_shapes=[
            vmem((1, 128)), vmem((1, 128)), vmem((128, 128)), vmem((1, 128)),
            vmem((512, 128)), vmem((1, 512)),
            vmem((128, _D)), vmem((1, 128)), vmem((128, 128)), vmem((1, 128)),
            vmem((128, 128)), vmem((1, 128)), vmem((128, 128)), vmem((1, 128)),
            vmem((_K, 128)),
        ],
    )(x, eps,
      W_e0.reshape(1, 10), b_e0.reshape(1, 10), W_e1, b_e1.reshape(1, 50),
      W_mu, b_mu.reshape(1, _D), W_lv, b_lv.reshape(1, _D),
      emb, e2, jnp.asarray(_ADJ),
      W_d, b_d.reshape(1, 100), W_d0, b_d0.reshape(1, 60),
      W_d1, b_d1.reshape(1, 30), W_d2, b_d2.reshape(1, 1))

    return ze, zq, de, dq, cs[0, 0], ss[0, 0]
